# baseline (speedup 1.0000x reference)
import functools

import jax
import jax.numpy as jnp
from jax import lax
from jax.experimental import pallas as pl
from jax.experimental.pallas import tpu as pltpu
from jax.experimental.pallas import tpu_sc as plsc


@functools.lru_cache(maxsize=None)
def _build_bcast_kernel(batch, seq_len, d_model, dtype_name):
    dtype = jnp.dtype(dtype_name)
    info = plsc.get_sparse_core_info()
    num_cores, num_subcores = info.num_cores, info.num_subcores
    num_workers = num_cores * num_subcores
    assert seq_len % num_workers == 0, seq_len
    rows_per_w = seq_len // num_workers

    mesh = plsc.VectorSubcoreMesh(core_axis_name="c", subcore_axis_name="s")

    n_chunks = 4
    assert rows_per_w % n_chunks == 0
    chunk = rows_per_w // n_chunks

    @functools.partial(
        pl.kernel,
        mesh=mesh,
        out_type=jax.ShapeDtypeStruct((batch, seq_len, d_model), dtype),
        scratch_types=[
            pltpu.VMEM((2, seq_len // num_workers // n_chunks, d_model), dtype),
            pltpu.SemaphoreType.DMA,
            pltpu.SemaphoreType.DMA,
        ],
    )
    def bcast(table_hbm, out_hbm, buf_v, rsem, wsem):
        wid = lax.axis_index("s") * num_cores + lax.axis_index("c")
        base = wid * rows_per_w
        reads = [
            pltpu.make_async_copy(
                table_hbm.at[pl.ds(base + j * chunk, chunk)],
                buf_v.at[j % 2],
                rsem,
            )
            for j in range(n_chunks)
        ]
        writes = [
            [
                pltpu.make_async_copy(
                    buf_v.at[j % 2],
                    out_hbm.at[b, pl.ds(base + j * chunk, chunk)],
                    wsem,
                )
                for b in range(batch)
            ]
            for j in range(n_chunks)
        ]
        reads[0].start()
        for j in range(n_chunks):
            reads[j].wait()
            for w in writes[j]:
                w.start()
            if j + 1 < n_chunks:
                if j >= 1:
                    for w in writes[j - 1]:
                        w.wait()
                reads[j + 1].start()
        for w in writes[n_chunks - 2]:
            w.wait()
        for w in writes[n_chunks - 1]:
            w.wait()

    return bcast


def kernel(x, pos_table):
    batch, seq_len, d_model = x.shape
    fn = _build_bcast_kernel(batch, seq_len, d_model, str(pos_table.dtype))
    return fn(pos_table[:seq_len])
